# baseline (speedup 1.0000x reference)
import functools

import jax
import jax.numpy as jnp
from jax import lax
from jax.experimental import pallas as pl
from jax.experimental.pallas import tpu as pltpu
from jax.experimental.pallas import tpu_sc as plsc

EMBED_D = 32
HALF_D = 16
VOCAB_ROWS = 100000
NC = 2
NS = 16
CH = 512
ROWS_PER_TILE = VOCAB_ROWS // NS


@jax.jit
def _sc_gather(idx_flat, table_pad):
    n = idx_flat.shape[0]
    b_per_w = n // NS
    n_ch = b_per_w // CH
    n_pairs = n_ch // 2
    mesh = plsc.VectorSubcoreMesh(core_axis_name="c", subcore_axis_name="s")

    @functools.partial(
        pl.kernel,
        mesh=mesh,
        out_type=jax.ShapeDtypeStruct((n, EMBED_D), jnp.float32),
        scratch_types=[
            pltpu.VMEM((CH,), jnp.int32),
            pltpu.VMEM((CH,), jnp.int32),
            pltpu.VMEM((CH, HALF_D), jnp.float32),
            pltpu.VMEM((CH, HALF_D), jnp.float32),
            pltpu.VMEM_SHARED((VOCAB_ROWS, HALF_D), jnp.float32),
            pltpu.SemaphoreType.DMA,
            pltpu.SemaphoreType.DMA,
        ],
        compiler_params=pltpu.CompilerParams(use_tc_tiling_on_sc=False),
    )
    def k(table_hbm, idx_hbm, out_hbm, idx0, idx1, rows0, rows1, tab_sp,
          sem0, sem1):
        sid = lax.axis_index("s")
        cid = lax.axis_index("c")
        base = sid * b_per_w
        col = cid * HALF_D

        toff = sid * ROWS_PER_TILE
        pltpu.sync_copy(
            table_hbm.at[pl.ds(toff, ROWS_PER_TILE), pl.ds(col, HALF_D)],
            tab_sp.at[pl.ds(toff, ROWS_PER_TILE)])
        plsc.subcore_barrier()

        pltpu.sync_copy(idx_hbm.at[pl.ds(base, CH)], idx0)
        pltpu.async_copy(tab_sp.at[idx0], rows0, sem0)

        def body(h, carry):
            g0 = 2 * h
            off0 = base + g0 * CH
            off1 = off0 + CH

            pltpu.sync_copy(idx_hbm.at[pl.ds(off1, CH)], idx1)
            pltpu.async_copy(tab_sp.at[idx1], rows1, sem1)

            pltpu.make_async_copy(tab_sp.at[idx0], rows0, sem0).wait()
            pltpu.sync_copy(rows0,
                            out_hbm.at[pl.ds(off0, CH), pl.ds(col, HALF_D)])

            @pl.when(h < n_pairs - 1)
            def _():
                off2 = off1 + CH
                pltpu.sync_copy(idx_hbm.at[pl.ds(off2, CH)], idx0)
                pltpu.async_copy(tab_sp.at[idx0], rows0, sem0)

            pltpu.make_async_copy(tab_sp.at[idx1], rows1, sem1).wait()
            pltpu.sync_copy(rows1,
                            out_hbm.at[pl.ds(off1, CH), pl.ds(col, HALF_D)])
            return carry

        lax.fori_loop(0, n_pairs, body, 0)

    return k(table_pad, idx_flat)


def kernel(x, table):
    B, P, L = x.shape
    zf, zi = lax.optimization_barrier((jnp.float32(0.0), jnp.int32(0)))
    out = _sc_gather(x.reshape(-1) + zi, table + zf)
    return (out + zf).reshape(B, P, L, EMBED_D)


# speedup vs baseline: 1.3110x; 1.3110x over previous
import functools

import jax
import jax.numpy as jnp
from jax import lax
from jax.experimental import pallas as pl
from jax.experimental.pallas import tpu as pltpu
from jax.experimental.pallas import tpu_sc as plsc

EMBED_D = 32
NC = 2
NS = 16
NW = NC * NS
CH = 1600
NSUB = 4
SUB = CH // NSUB


@jax.jit
def _sc_gather(idx_flat, table):
    n = idx_flat.shape[0]
    b_per_w = n // NW
    n_ch = b_per_w // CH
    n_pairs = n_ch // 2
    mesh = plsc.VectorSubcoreMesh(core_axis_name="c", subcore_axis_name="s")

    @functools.partial(
        pl.kernel,
        mesh=mesh,
        out_type=jax.ShapeDtypeStruct((n, EMBED_D), jnp.float32),
        scratch_types=[
            pltpu.VMEM((CH,), jnp.int32),
            pltpu.VMEM((CH,), jnp.int32),
            pltpu.VMEM((CH, EMBED_D), jnp.float32),
            pltpu.VMEM((CH, EMBED_D), jnp.float32),
            pltpu.SemaphoreType.DMA,
            pltpu.SemaphoreType.DMA,
        ],
        compiler_params=pltpu.CompilerParams(use_tc_tiling_on_sc=False),
    )
    def k(table_hbm, idx_hbm, out_hbm, idx0, idx1, rows0, rows1, sem0, sem1):
        wid = lax.axis_index("s") * NC + lax.axis_index("c")
        base = wid * b_per_w

        def fire(idx_v, rows_v, sem):
            for j in range(NSUB):
                s = pl.ds(j * SUB, SUB)
                pltpu.async_copy(table_hbm.at[idx_v.at[s]], rows_v.at[s], sem)

        def drain(idx_v, rows_v, sem):
            for j in range(NSUB):
                s = pl.ds(j * SUB, SUB)
                pltpu.make_async_copy(table_hbm.at[idx_v.at[s]], rows_v.at[s], sem).wait()

        pltpu.sync_copy(idx_hbm.at[pl.ds(base, CH)], idx0)
        fire(idx0, rows0, sem0)

        def body(h, carry):
            g0 = 2 * h
            off0 = base + g0 * CH
            off1 = off0 + CH

            pltpu.sync_copy(idx_hbm.at[pl.ds(off1, CH)], idx1)
            fire(idx1, rows1, sem1)

            drain(idx0, rows0, sem0)
            pltpu.sync_copy(rows0, out_hbm.at[pl.ds(off0, CH)])

            @pl.when(h < n_pairs - 1)
            def _():
                off2 = off1 + CH
                pltpu.sync_copy(idx_hbm.at[pl.ds(off2, CH)], idx0)
                fire(idx0, rows0, sem0)

            drain(idx1, rows1, sem1)
            pltpu.sync_copy(rows1, out_hbm.at[pl.ds(off1, CH)])
            return carry

        lax.fori_loop(0, n_pairs, body, 0)

    return k(table, idx_flat)


def kernel(x, table):
    B, P, L = x.shape
    out = _sc_gather(x.reshape(-1), table)
    return out.reshape(B, P, L, EMBED_D)
